# Initial kernel scaffold; baseline (speedup 1.0000x reference)
import jax
import jax.numpy as jnp
from jax.experimental import pallas as pl


def kernel(h, prototypes, evidence, W_QR, W_KR, W_out):
    raise NotImplementedError("write your pallas kernel here")


# speedup vs baseline: 3.4033x; 3.4033x over previous
import functools

import jax
import jax.numpy as jnp
from jax.experimental import pallas as pl

DIM = 512
NUM_PROTOS = 100000
TOP_K = 16
CHUNK = 64
NUM_HEADS = 4
HEAD_DIM = DIM // NUM_HEADS
TEMPERATURE = 0.1
BATCH = 4096

ROWS_BLK = 4096
CHUNKS_BLK = ROWS_BLK // CHUNK
G1 = (NUM_PROTOS + ROWS_BLK - 1) // ROWS_BLK
PC_PAD = G1 * CHUNKS_BLK
PC = (NUM_PROTOS + CHUNK - 1) // CHUNK
B_BLK = 256
NEG = -1e30
F32 = jnp.float32
BF16 = jnp.bfloat16
HI = jax.lax.Precision.HIGHEST


def _nt(a, b):
    return jax.lax.dot_general(a, b, (((1,), (1,)), ((), ())),
                               preferred_element_type=F32)


def _nn(a, b, precision=None):
    return jax.lax.dot_general(a, b, (((1,), (0,)), ((), ())),
                               preferred_element_type=F32,
                               precision=precision)


def _head_norm(x):
    x2 = x * x
    r = jax.lax.broadcasted_iota(jnp.int32, (DIM, DIM), 0) // HEAD_DIM
    c = jax.lax.broadcasted_iota(jnp.int32, (DIM, DIM), 1) // HEAD_DIM
    mm = (r == c).astype(F32)
    hs = _nn(x2, mm, precision=HI)
    return x / jnp.maximum(jnp.sqrt(hs), 1e-12)


def _k1_body(p_ref, ev_ref, wkrt_ref, woutt_ref, kn_ref, cw_ref):
    i = pl.program_id(0)
    p = p_ref[...]
    rows_left = NUM_PROTOS - i * ROWS_BLK
    rid = jax.lax.broadcasted_iota(jnp.int32, (ROWS_BLK, 1), 0)
    p = jnp.where(rid < rows_left, p, 0.0)
    ev = ev_ref[...] + 1e-8
    w = ev / jnp.sum(ev, axis=1, keepdims=True)
    p3 = p.reshape(CHUNKS_BLK, CHUNK, DIM)
    raw = jnp.sum(p3 * w[:, :, None], axis=1)
    rr = jnp.sum(raw * raw, axis=1, keepdims=True)
    comp = raw / jnp.maximum(jnp.sqrt(rr), 1e-12)
    compb = comp.astype(BF16)
    kr = _nn(compb, wkrt_ref[...])
    kn_ref[...] = _head_norm(kr).astype(BF16)
    cw_ref[...] = _nn(compb, woutt_ref[...])


def _k2_body(h_ref, wqrt_ref, kn_ref, cw_ref, out_ref, idx_ref, wts_ref):
    qr = _nn(h_ref[...], wqrt_ref[...])
    qnb = _head_norm(qr).astype(BF16)
    scores = _nt(qnb, kn_ref[...]) * (1.0 / (NUM_HEADS * TEMPERATURE))
    col = jax.lax.broadcasted_iota(jnp.int32, (B_BLK, PC_PAD), 1)
    scores = jnp.where(col < PC, scores, NEG)

    k16 = jax.lax.broadcasted_iota(jnp.int32, (B_BLK, TOP_K), 1)
    idx_mat = jnp.zeros((B_BLK, TOP_K), jnp.int32)
    e_mat = jnp.zeros((B_BLK, TOP_K), F32)
    u = jnp.zeros((B_BLK, PC_PAD), F32)
    m0 = None
    for k in range(TOP_K):
        m = jnp.max(scores, axis=1, keepdims=True)
        if k == 0:
            m0 = m
        hit = scores == m
        idx = jnp.min(jnp.where(hit, col, PC_PAD), axis=1, keepdims=True)
        one = col == idx
        e = jnp.exp(m - m0)
        u = u + jnp.where(one, e, 0.0)
        scores = jnp.where(one, NEG, scores)
        idx_mat = jnp.where(k16 == k, idx, idx_mat)
        e_mat = jnp.where(k16 == k, e, e_mat)
    z = jnp.sum(e_mat, axis=1, keepdims=True)
    rz = 1.0 / z
    ret = _nn(u, cw_ref[...], precision=HI)
    out_ref[...] = ret * rz
    idx_ref[...] = idx_mat
    wts_ref[...] = e_mat * rz


@functools.partial(jax.jit, static_argnames=("interpret",))
def kernel(h, prototypes, evidence, W_QR, W_KR, W_out, interpret=False):
    evf = evidence.astype(F32)
    ev2 = jnp.pad(evf, (0, PC_PAD * CHUNK - NUM_PROTOS)).reshape(PC_PAD, CHUNK)
    hb = h.astype(BF16)
    wqrt = W_QR.T.astype(BF16)
    wkrt = W_KR.T.astype(BF16)
    woutt = W_out.T.astype(BF16)

    kn, cw = pl.pallas_call(
        _k1_body,
        grid=(G1,),
        in_specs=[
            pl.BlockSpec((ROWS_BLK, DIM), lambda i: (i, 0)),
            pl.BlockSpec((CHUNKS_BLK, CHUNK), lambda i: (i, 0)),
            pl.BlockSpec((DIM, DIM), lambda i: (0, 0)),
            pl.BlockSpec((DIM, DIM), lambda i: (0, 0)),
        ],
        out_specs=[
            pl.BlockSpec((CHUNKS_BLK, DIM), lambda i: (i, 0)),
            pl.BlockSpec((CHUNKS_BLK, DIM), lambda i: (i, 0)),
        ],
        out_shape=[
            jax.ShapeDtypeStruct((PC_PAD, DIM), BF16),
            jax.ShapeDtypeStruct((PC_PAD, DIM), F32),
        ],
        interpret=interpret,
    )(prototypes, ev2, wkrt, woutt)

    retrieved, topk_idx, topk_wts = pl.pallas_call(
        _k2_body,
        grid=(BATCH // B_BLK,),
        in_specs=[
            pl.BlockSpec((B_BLK, DIM), lambda i: (i, 0)),
            pl.BlockSpec((DIM, DIM), lambda i: (0, 0)),
            pl.BlockSpec((PC_PAD, DIM), lambda i: (0, 0)),
            pl.BlockSpec((PC_PAD, DIM), lambda i: (0, 0)),
        ],
        out_specs=[
            pl.BlockSpec((B_BLK, DIM), lambda i: (i, 0)),
            pl.BlockSpec((B_BLK, TOP_K), lambda i: (i, 0)),
            pl.BlockSpec((B_BLK, TOP_K), lambda i: (i, 0)),
        ],
        out_shape=[
            jax.ShapeDtypeStruct((BATCH, DIM), F32),
            jax.ShapeDtypeStruct((BATCH, TOP_K), jnp.int32),
            jax.ShapeDtypeStruct((BATCH, TOP_K), F32),
        ],
        interpret=interpret,
    )(hb, wqrt, kn, cw)

    return retrieved, topk_idx, topk_wts


# speedup vs baseline: 4.0335x; 1.1852x over previous
import functools

import jax
import jax.numpy as jnp
from jax.experimental import pallas as pl

DIM = 512
NUM_PROTOS = 100000
TOP_K = 16
CHUNK = 64
NUM_HEADS = 4
HEAD_DIM = DIM // NUM_HEADS
TEMPERATURE = 0.1
BATCH = 4096

ROWS_BLK = 4096
CHUNKS_BLK = ROWS_BLK // CHUNK
G1 = (NUM_PROTOS + ROWS_BLK - 1) // ROWS_BLK
PC_PAD = G1 * CHUNKS_BLK
PC = (NUM_PROTOS + CHUNK - 1) // CHUNK
B_BLK = 256
NEG = -1e30
F32 = jnp.float32
BF16 = jnp.bfloat16
HI = jax.lax.Precision.HIGHEST


def _nt(a, b):
    return jax.lax.dot_general(a, b, (((1,), (1,)), ((), ())),
                               preferred_element_type=F32)


def _nn(a, b, precision=None):
    return jax.lax.dot_general(a, b, (((1,), (0,)), ((), ())),
                               preferred_element_type=F32,
                               precision=precision)


def _head_norm(x):
    x2 = x * x
    parts = []
    for hd in range(NUM_HEADS):
        hs = jnp.sum(x2[:, hd * HEAD_DIM:(hd + 1) * HEAD_DIM],
                     axis=1, keepdims=True)
        inv = 1.0 / jnp.maximum(jnp.sqrt(hs), 1e-12)
        parts.append(jnp.broadcast_to(inv, (x.shape[0], HEAD_DIM)))
    return x * jnp.concatenate(parts, axis=1)


def _k1_body(p_ref, ev_ref, wkrt_ref, woutt_ref, kn_ref, cw_ref):
    i = pl.program_id(0)
    p = p_ref[...]
    rows_left = NUM_PROTOS - i * ROWS_BLK
    rid = jax.lax.broadcasted_iota(jnp.int32, (ROWS_BLK, 1), 0)
    p = jnp.where(rid < rows_left, p, 0.0)
    ev = ev_ref[...] + 1e-8
    w = ev / jnp.sum(ev, axis=1, keepdims=True)
    p3 = p.reshape(CHUNKS_BLK, CHUNK, DIM)
    raw = jnp.sum(p3 * w[:, :, None], axis=1)
    rr = jnp.sum(raw * raw, axis=1, keepdims=True)
    comp = raw / jnp.maximum(jnp.sqrt(rr), 1e-12)
    compb = comp.astype(BF16)
    kr = _nn(compb, wkrt_ref[...])
    kn_ref[...] = _head_norm(kr).astype(BF16)
    cw_ref[...] = _nn(compb, woutt_ref[...]).astype(BF16)


def _k2_body(h_ref, wqrt_ref, kn_ref, cw_ref, out_ref, idx_ref, wts_ref):
    qr = _nn(h_ref[...], wqrt_ref[...])
    qnb = _head_norm(qr).astype(BF16)
    scores = _nt(qnb, kn_ref[...]) * (1.0 / (NUM_HEADS * TEMPERATURE))
    col = jax.lax.broadcasted_iota(jnp.int32, (B_BLK, PC_PAD), 1)
    scores = jnp.where(col < PC, scores, NEG)

    k16 = jax.lax.broadcasted_iota(jnp.int32, (B_BLK, TOP_K), 1)
    idx_mat = jnp.zeros((B_BLK, TOP_K), jnp.int32)
    e_mat = jnp.zeros((B_BLK, TOP_K), F32)
    u = jnp.zeros((B_BLK, PC_PAD), F32)
    m0 = None
    for k in range(TOP_K):
        m = jnp.max(scores, axis=1, keepdims=True)
        if k == 0:
            m0 = m
        cand = jnp.where(scores == m, col, PC_PAD)
        idx = jnp.min(cand, axis=1, keepdims=True)
        sel = cand == idx
        e = jnp.exp(m - m0)
        u = u + jnp.where(sel, e, 0.0)
        scores = jnp.where(sel, NEG, scores)
        idx_mat = jnp.where(k16 == k, idx, idx_mat)
        e_mat = jnp.where(k16 == k, e, e_mat)
    z = jnp.sum(e_mat, axis=1, keepdims=True)
    rz = 1.0 / z
    u_hi = u.astype(BF16)
    u_lo = (u - u_hi.astype(F32)).astype(BF16)
    cwb = cw_ref[...]
    ret = _nn(u_hi, cwb) + _nn(u_lo, cwb)
    out_ref[...] = ret * rz
    idx_ref[...] = idx_mat
    wts_ref[...] = e_mat * rz


@functools.partial(jax.jit, static_argnames=("interpret",))
def kernel(h, prototypes, evidence, W_QR, W_KR, W_out, interpret=False):
    evf = evidence.astype(F32)
    ev2 = jnp.pad(evf, (0, PC_PAD * CHUNK - NUM_PROTOS)).reshape(PC_PAD, CHUNK)
    hb = h.astype(BF16)
    wqrt = W_QR.T.astype(BF16)
    wkrt = W_KR.T.astype(BF16)
    woutt = W_out.T.astype(BF16)

    kn, cw = pl.pallas_call(
        _k1_body,
        grid=(G1,),
        in_specs=[
            pl.BlockSpec((ROWS_BLK, DIM), lambda i: (i, 0)),
            pl.BlockSpec((CHUNKS_BLK, CHUNK), lambda i: (i, 0)),
            pl.BlockSpec((DIM, DIM), lambda i: (0, 0)),
            pl.BlockSpec((DIM, DIM), lambda i: (0, 0)),
        ],
        out_specs=[
            pl.BlockSpec((CHUNKS_BLK, DIM), lambda i: (i, 0)),
            pl.BlockSpec((CHUNKS_BLK, DIM), lambda i: (i, 0)),
        ],
        out_shape=[
            jax.ShapeDtypeStruct((PC_PAD, DIM), BF16),
            jax.ShapeDtypeStruct((PC_PAD, DIM), BF16),
        ],
        interpret=interpret,
    )(prototypes, ev2, wkrt, woutt)

    retrieved, topk_idx, topk_wts = pl.pallas_call(
        _k2_body,
        grid=(BATCH // B_BLK,),
        in_specs=[
            pl.BlockSpec((B_BLK, DIM), lambda i: (i, 0)),
            pl.BlockSpec((DIM, DIM), lambda i: (0, 0)),
            pl.BlockSpec((PC_PAD, DIM), lambda i: (0, 0)),
            pl.BlockSpec((PC_PAD, DIM), lambda i: (0, 0)),
        ],
        out_specs=[
            pl.BlockSpec((B_BLK, DIM), lambda i: (i, 0)),
            pl.BlockSpec((B_BLK, TOP_K), lambda i: (i, 0)),
            pl.BlockSpec((B_BLK, TOP_K), lambda i: (i, 0)),
        ],
        out_shape=[
            jax.ShapeDtypeStruct((BATCH, DIM), F32),
            jax.ShapeDtypeStruct((BATCH, TOP_K), jnp.int32),
            jax.ShapeDtypeStruct((BATCH, TOP_K), F32),
        ],
        interpret=interpret,
    )(hb, wqrt, kn, cw)

    return retrieved, topk_idx, topk_wts
